# Initial kernel scaffold; baseline (speedup 1.0000x reference)
import jax
import jax.numpy as jnp
from jax.experimental import pallas as pl


def kernel(boxes, scores):
    raise NotImplementedError("write your pallas kernel here")


# speedup vs baseline: 8.9366x; 8.9366x over previous
import jax
import jax.numpy as jnp
from jax.experimental import pallas as pl
from jax.experimental.pallas import tpu as pltpu

_N = 20000
_PRE_K = 1000
_POST_K = 100
_SCORE_THRESH = 0.05
_NMS_THRESH = 0.6
_P = 1024
_NEG = -3.0e38


def _nms_kernel(boxes_ref, boxest_ref, scol_ref, srow_ref, out_ref, iou_s):
    x1c = boxes_ref[:, 0:1]
    y1c = boxes_ref[:, 1:2]
    x2c = boxes_ref[:, 2:3]
    y2c = boxes_ref[:, 3:4]
    x1r = boxest_ref[0:1, :]
    y1r = boxest_ref[1:2, :]
    x2r = boxest_ref[2:3, :]
    y2r = boxest_ref[3:4, :]

    area_c = jnp.maximum(x2c - x1c, 0.0) * jnp.maximum(y2c - y1c, 0.0)
    area_r = jnp.maximum(x2r - x1r, 0.0) * jnp.maximum(y2r - y1r, 0.0)
    iw = jnp.maximum(jnp.minimum(x2c, x2r) - jnp.maximum(x1c, x1r), 0.0)
    ih = jnp.maximum(jnp.minimum(y2c, y2r) - jnp.maximum(y1c, y1r), 0.0)
    inter = iw * ih
    union = area_c + area_r - inter
    iou_s[:, :] = inter / jnp.maximum(union, 1e-9)

    lane = jax.lax.broadcasted_iota(jnp.int32, (1, _P), 1)

    def body(i, keep):
        row = iou_s[pl.ds(i, 1), :]
        keep_i = jnp.sum(jnp.where(lane == i, keep, 0.0))
        sup = (row > _NMS_THRESH) & (lane > i) & (keep_i > 0.5)
        return jnp.where(sup, 0.0, keep)

    keep = jax.lax.fori_loop(0, _PRE_K, body, jnp.ones((1, _P), jnp.float32))

    srow = srow_ref[0:1, :]
    k_row = keep * (srow > _NEG).astype(jnp.float32)

    sub = jax.lax.broadcasted_iota(jnp.int32, (_P, _P), 0)
    lan = jax.lax.broadcasted_iota(jnp.int32, (_P, _P), 1)
    tri = (sub <= lan).astype(jnp.float32)
    cum_k = jax.lax.dot_general(
        k_row, tri, (((1,), (0,)), ((), ())),
        preferred_element_type=jnp.float32)
    nk_row = 1.0 - k_row
    cum_nk = jax.lax.dot_general(
        nk_row, tri, (((1,), (0,)), ((), ())),
        preferred_element_type=jnp.float32)
    num_k = jnp.sum(k_row)

    slot = jnp.where(k_row > 0.5, cum_k - 1.0, num_k + cum_nk - 1.0)

    out_row = jax.lax.broadcasted_iota(jnp.int32, (128, 1), 0).astype(jnp.float32)
    sel = (out_row == slot).astype(jnp.float32)

    eye = (sub == lan).astype(jnp.float32)
    k_col = jax.lax.dot_general(
        eye, k_row, (((1,), (1,)), ((), ())),
        preferred_element_type=jnp.float32)
    s_keep = jnp.where(k_col > 0.5, scol_ref[:, :], 0.0)
    zeros3 = jnp.zeros((_P, 3), jnp.float32)
    data = jnp.concatenate([boxes_ref[:, :], s_keep, zeros3], axis=1)
    out_ref[:, :] = jax.lax.dot_general(
        sel, data, (((1,), (0,)), ((), ())),
        preferred_element_type=jnp.float32)


def kernel(boxes, scores):
    masked = jnp.where(scores > _SCORE_THRESH, scores, -jnp.inf)
    top_scores, top_idx = jax.lax.top_k(masked, _PRE_K)
    top_boxes = jnp.take(boxes, top_idx, axis=0)

    pad = _P - _PRE_K
    boxes_p = jnp.concatenate([top_boxes, jnp.zeros((pad, 4), jnp.float32)], axis=0)
    scores_p = jnp.concatenate(
        [top_scores, jnp.full((pad,), -jnp.inf, jnp.float32)], axis=0)
    boxest_p = boxes_p.T
    scol = scores_p[:, None]
    srow = scores_p[None, :]

    out = pl.pallas_call(
        _nms_kernel,
        out_shape=jax.ShapeDtypeStruct((128, 8), jnp.float32),
        scratch_shapes=[pltpu.VMEM((_P, _P), jnp.float32)],
    )(boxes_p, boxest_p, scol, srow)
    return out[:_POST_K, :5]
